# baseline (speedup 1.0000x reference)
import functools

import jax
import jax.numpy as jnp
from jax import lax
from jax.experimental import pallas as pl
from jax.experimental.pallas import tpu as pltpu
from jax.experimental.pallas import tpu_sc as plsc

_B = 4096
_ND = 13
_NS = 26
_F = _ND + _NS
_V = 100000
_E = 16
_MAXN = 0.1
_BNS = _B * _NS
_NW = 32
_BPW = _BNS // _NW
_BK = 512
_SD = _ND * _E
_SS = _NS * _E


def _sc_gather(fo_tab, so_tab, idx):
    mesh = plsc.VectorSubcoreMesh(core_axis_name="c", subcore_axis_name="s")
    out_t = jax.ShapeDtypeStruct((_BNS, _E), jnp.float32)

    @functools.partial(
        pl.kernel,
        mesh=mesh,
        out_type=(out_t, out_t),
        compiler_params=pltpu.CompilerParams(use_tc_tiling_on_sc=False),
        scratch_types=[
            pltpu.VMEM((_BPW,), jnp.int32),
            pltpu.VMEM((_BPW, _E), jnp.float32),
            pltpu.VMEM((_BPW, _E), jnp.float32),
            pltpu.SemaphoreType.DMA,
            pltpu.SemaphoreType.DMA,
        ],
    )
    def k(fo_hbm, so_hbm, idx_hbm, fo_out, so_out, idx_v, fo_v, so_v, s1, s2):
        wid = lax.axis_index("s") * 2 + lax.axis_index("c")
        base = wid * _BPW
        pltpu.sync_copy(idx_hbm.at[pl.ds(base, _BPW)], idx_v)
        c1 = pltpu.async_copy(fo_hbm.at[idx_v], fo_v, s1)
        c2 = pltpu.async_copy(so_hbm.at[idx_v], so_v, s2)
        c1.wait()
        c2.wait()
        pltpu.sync_copy(fo_v, fo_out.at[pl.ds(base, _BPW)])
        pltpu.sync_copy(so_v, so_out.at[pl.ds(base, _BPW)])

    return k(fo_tab, so_tab, idx)


def _tc_body(xd_ref, fo_g_ref, so_g_ref,
             fo_dw_ref, fo_db_ref, fo_lw_ref, fo_lb_ref,
             so_dw_ref, so_db_ref,
             w1d_ref, w1s_ref, b1_ref, w2_ref, b2_ref, w3t_ref, b3_ref,
             msum_ref, mexp_ref, rexp_ref,
             row_out_ref, so_out_ref,
             cs_d, cs_s, ss_acc):
    i = pl.program_id(0)

    @pl.when(i == 0)
    def _():
        cs_d[...] = jnp.zeros_like(cs_d)
        cs_s[...] = jnp.zeros_like(cs_s)
        ss_acc[...] = jnp.zeros_like(ss_acc)

    xd = xd_ref[...]
    fo_g = fo_g_ref[...]
    so_g = so_g_ref[...]
    msum = msum_ref[...]
    mexp = mexp_ref[...]
    rexp = rexp_ref[...]

    def renorm_scale(g):
        sq = jnp.dot(g * g, msum, preferred_element_type=jnp.float32)
        n = jnp.sqrt(sq)
        return jnp.where(n > _MAXN, _MAXN / (n + 1e-7), 1.0)

    fo_scale = renorm_scale(fo_g)
    so_scale = renorm_scale(so_g)

    fo_dw_sum = jnp.sum(fo_dw_ref[...], axis=1)[None, :]
    fo_db_sum = jnp.sum(fo_db_ref[...], axis=1)[None, :]
    fo_d = xd * fo_dw_sum + fo_db_sum
    fo_rowsum = jnp.dot(fo_g, msum, preferred_element_type=jnp.float32)
    fo_s = fo_scale * fo_rowsum
    lw = fo_lw_ref[...]
    first = (jnp.sum(fo_d * lw[:, :_ND], axis=1, keepdims=True)
             + jnp.sum(fo_s * lw[:, _ND:], axis=1, keepdims=True)
             + fo_lb_ref[0, 0])

    so_d = jnp.dot(xd, rexp, preferred_element_type=jnp.float32)
    so_d = so_d * so_dw_ref[...] + so_db_ref[...]
    so_s = so_g * jnp.dot(so_scale, mexp, preferred_element_type=jnp.float32)

    cs_d[...] += jnp.sum(so_d, axis=0, keepdims=True)
    cs_s[...] += jnp.sum(so_s, axis=0, keepdims=True)
    ss_acc[...] += (jnp.sum(so_d * so_d, keepdims=True)[:, :1]
                    + jnp.sum(so_s * so_s, keepdims=True)[:, :1])

    h = (jnp.dot(so_d, w1d_ref[...], preferred_element_type=jnp.float32)
         + jnp.dot(so_s, w1s_ref[...], preferred_element_type=jnp.float32)
         + b1_ref[...])
    h = jnp.maximum(h, 0.0)
    h = jnp.dot(h, w2_ref[...], preferred_element_type=jnp.float32) + b2_ref[...]
    h = jnp.maximum(h, 0.0)
    deep = jnp.sum(h * w3t_ref[...], axis=1, keepdims=True) + b3_ref[0, 0]

    row_out_ref[...] = first + deep

    csd = cs_d[...]
    css = cs_s[...]
    so_out_ref[...] = 0.5 * (jnp.sum(csd * csd, keepdims=True)[:, :1]
                             + jnp.sum(css * css, keepdims=True)[:, :1]
                             - ss_acc[...])


def kernel(X, fo_dense_W, fo_dense_b, fo_tables, fo_lin_W, fo_lin_b,
           so_dense_W, so_dense_b, so_tables, W1, b1, W2, b2, W3, b3):
    f32 = jnp.float32
    Xd = X[:, :_ND].astype(f32)
    Xs = X[:, _ND:]
    offs = (jnp.arange(_NS, dtype=jnp.int32) * _V)[None, :]
    flat_idx = (Xs + offs).reshape(-1)

    fo_g, so_g = _sc_gather(fo_tables.reshape(_NS * _V, _E),
                            so_tables.reshape(_NS * _V, _E), flat_idx)
    fo_g = fo_g.reshape(_B, _SS)
    so_g = so_g.reshape(_B, _SS)

    msum = jnp.kron(jnp.eye(_NS, dtype=f32), jnp.ones((_E, 1), f32))
    mexp = msum.T
    rexp = jnp.kron(jnp.eye(_ND, dtype=f32), jnp.ones((1, _E), f32))

    nblk = _B // _BK
    whole = lambda shape: pl.BlockSpec(shape, lambda i: (0, 0))
    grid_spec = dict(
        grid=(nblk,),
        in_specs=[
            pl.BlockSpec((_BK, _ND), lambda i: (i, 0)),
            pl.BlockSpec((_BK, _SS), lambda i: (i, 0)),
            pl.BlockSpec((_BK, _SS), lambda i: (i, 0)),
            whole((_ND, _E)), whole((_ND, _E)),
            whole((1, _F)), whole((1, 1)),
            whole((1, _SD)), whole((1, _SD)),
            whole((_SD, 400)), whole((_SS, 400)), whole((1, 400)),
            whole((400, 400)), whole((1, 400)),
            whole((1, 400)), whole((1, 1)),
            whole((_SS, _NS)), whole((_NS, _SS)), whole((_ND, _SD)),
        ],
        out_specs=[
            pl.BlockSpec((_BK, 1), lambda i: (i, 0)),
            pl.BlockSpec((1, 1), lambda i: (0, 0)),
        ],
    )
    rows, so_scalar = pl.pallas_call(
        _tc_body,
        out_shape=[jax.ShapeDtypeStruct((_B, 1), f32),
                   jax.ShapeDtypeStruct((1, 1), f32)],
        scratch_shapes=[pltpu.VMEM((1, _SD), f32),
                        pltpu.VMEM((1, _SS), f32),
                        pltpu.VMEM((1, 1), f32)],
        **grid_spec,
    )(Xd, fo_g, so_g,
      fo_dense_W, fo_dense_b, fo_lin_W, fo_lin_b.reshape(1, 1),
      so_dense_W.reshape(1, _SD), so_dense_b.reshape(1, _SD),
      W1[:_SD], W1[_SD:], b1.reshape(1, 400),
      W2, b2.reshape(1, 400),
      W3.reshape(1, 400), b3.reshape(1, 1),
      msum, mexp, rexp)

    return rows + so_scalar


# speedup vs baseline: 1.0019x; 1.0019x over previous
import functools

import jax
import jax.numpy as jnp
from jax import lax
from jax.experimental import pallas as pl
from jax.experimental.pallas import tpu as pltpu
from jax.experimental.pallas import tpu_sc as plsc

_B = 4096
_ND = 13
_NS = 26
_F = _ND + _NS
_V = 100000
_E = 16
_MAXN = 0.1
_BNS = _B * _NS
_NW = 32
_BPW = _BNS // _NW
_BK = 512
_SD = _ND * _E
_SS = _NS * _E


def _sc_gather(fo_tab, so_tab, idx):
    mesh = plsc.VectorSubcoreMesh(core_axis_name="c", subcore_axis_name="s")
    out_t = jax.ShapeDtypeStruct((_BNS, _E), jnp.float32)

    @functools.partial(
        pl.kernel,
        mesh=mesh,
        out_type=(out_t, out_t),
        compiler_params=pltpu.CompilerParams(use_tc_tiling_on_sc=False),
        scratch_types=[
            pltpu.VMEM((_BPW,), jnp.int32),
            pltpu.VMEM((_BPW, _E), jnp.float32),
            pltpu.VMEM((_BPW, _E), jnp.float32),
            pltpu.SemaphoreType.DMA,
            pltpu.SemaphoreType.DMA,
        ],
    )
    def k(fo_hbm, so_hbm, idx_hbm, fo_out, so_out, idx_v, fo_v, so_v, s1, s2):
        wid = lax.axis_index("s") * 2 + lax.axis_index("c")
        base = wid * _BPW
        fo_flat = fo_hbm
        so_flat = so_hbm
        pltpu.sync_copy(idx_hbm.at[pl.ds(base, _BPW)], idx_v)
        c1 = pltpu.async_copy(fo_flat.at[idx_v], fo_v, s1)
        c2 = pltpu.async_copy(so_flat.at[idx_v], so_v, s2)
        c1.wait()
        c2.wait()
        pltpu.sync_copy(fo_v, fo_out.at[pl.ds(base, _BPW)])
        pltpu.sync_copy(so_v, so_out.at[pl.ds(base, _BPW)])

    return k(fo_tab, so_tab, idx)


def _tc_body(xd_ref, fo_g_ref, so_g_ref,
             fo_dw_ref, fo_db_ref, fo_lw_ref, fo_lb_ref,
             so_dw_ref, so_db_ref,
             w1d_ref, w1s_ref, b1_ref, w2_ref, b2_ref, w3t_ref, b3_ref,
             msum_ref, mexp_ref, rexp_ref,
             row_out_ref, so_out_ref,
             cs_d, cs_s, ss_acc):
    i = pl.program_id(0)

    @pl.when(i == 0)
    def _():
        cs_d[...] = jnp.zeros_like(cs_d)
        cs_s[...] = jnp.zeros_like(cs_s)
        ss_acc[...] = jnp.zeros_like(ss_acc)

    xd = xd_ref[...]
    fo_g = fo_g_ref[...]
    so_g = so_g_ref[...]
    msum = msum_ref[...]
    mexp = mexp_ref[...]
    rexp = rexp_ref[...]

    def renorm_scale(g):
        sq = jnp.dot(g * g, msum, preferred_element_type=jnp.float32)
        n = jnp.sqrt(sq)
        return jnp.where(n > _MAXN, _MAXN / (n + 1e-7), 1.0)

    fo_scale = renorm_scale(fo_g)
    so_scale = renorm_scale(so_g)

    fo_dw_sum = jnp.sum(fo_dw_ref[...], axis=1)[None, :]
    fo_db_sum = jnp.sum(fo_db_ref[...], axis=1)[None, :]
    fo_d = xd * fo_dw_sum + fo_db_sum
    fo_rowsum = jnp.dot(fo_g, msum, preferred_element_type=jnp.float32)
    fo_s = fo_scale * fo_rowsum
    lw = fo_lw_ref[...]
    first = (jnp.sum(fo_d * lw[:, :_ND], axis=1, keepdims=True)
             + jnp.sum(fo_s * lw[:, _ND:], axis=1, keepdims=True)
             + fo_lb_ref[0, 0])

    so_d = jnp.dot(xd, rexp, preferred_element_type=jnp.float32)
    so_d = so_d * so_dw_ref[...] + so_db_ref[...]
    so_s = so_g * jnp.dot(so_scale, mexp, preferred_element_type=jnp.float32)

    cs_d[...] += jnp.sum(so_d, axis=0, keepdims=True)
    cs_s[...] += jnp.sum(so_s, axis=0, keepdims=True)
    ss_acc[...] += (jnp.sum(so_d * so_d, keepdims=True)[:, :1]
                    + jnp.sum(so_s * so_s, keepdims=True)[:, :1])

    h = (jnp.dot(so_d, w1d_ref[...], preferred_element_type=jnp.float32)
         + jnp.dot(so_s, w1s_ref[...], preferred_element_type=jnp.float32)
         + b1_ref[...])
    h = jnp.maximum(h, 0.0)
    h = jnp.dot(h, w2_ref[...], preferred_element_type=jnp.float32) + b2_ref[...]
    h = jnp.maximum(h, 0.0)
    deep = jnp.sum(h * w3t_ref[...], axis=1, keepdims=True) + b3_ref[0, 0]

    row_out_ref[...] = first + deep

    csd = cs_d[...]
    css = cs_s[...]
    so_out_ref[...] = 0.5 * (jnp.sum(csd * csd, keepdims=True)[:, :1]
                             + jnp.sum(css * css, keepdims=True)[:, :1]
                             - ss_acc[...])


def kernel(X, fo_dense_W, fo_dense_b, fo_tables, fo_lin_W, fo_lin_b,
           so_dense_W, so_dense_b, so_tables, W1, b1, W2, b2, W3, b3):
    f32 = jnp.float32
    Xd = X[:, :_ND].astype(f32)
    Xs = X[:, _ND:]
    offs = (jnp.arange(_NS, dtype=jnp.int32) * _V)[None, :]
    flat_idx = (Xs + offs).reshape(-1)

    fo_g, so_g = _sc_gather(fo_tables.reshape(_NS * _V, _E),
                            so_tables.reshape(_NS * _V, _E), flat_idx)
    fo_g = fo_g.reshape(_B, _SS)
    so_g = so_g.reshape(_B, _SS)

    msum = jnp.kron(jnp.eye(_NS, dtype=f32), jnp.ones((_E, 1), f32))
    mexp = msum.T
    rexp = jnp.kron(jnp.eye(_ND, dtype=f32), jnp.ones((1, _E), f32))

    nblk = _B // _BK
    whole = lambda shape: pl.BlockSpec(shape, lambda i: (0, 0))
    grid_spec = dict(
        grid=(nblk,),
        in_specs=[
            pl.BlockSpec((_BK, _ND), lambda i: (i, 0)),
            pl.BlockSpec((_BK, _SS), lambda i: (i, 0)),
            pl.BlockSpec((_BK, _SS), lambda i: (i, 0)),
            whole((_ND, _E)), whole((_ND, _E)),
            whole((1, _F)), whole((1, 1)),
            whole((1, _SD)), whole((1, _SD)),
            whole((_SD, 400)), whole((_SS, 400)), whole((1, 400)),
            whole((400, 400)), whole((1, 400)),
            whole((1, 400)), whole((1, 1)),
            whole((_SS, _NS)), whole((_NS, _SS)), whole((_ND, _SD)),
        ],
        out_specs=[
            pl.BlockSpec((_BK, 1), lambda i: (i, 0)),
            pl.BlockSpec((1, 1), lambda i: (0, 0)),
        ],
    )
    rows, so_scalar = pl.pallas_call(
        _tc_body,
        out_shape=[jax.ShapeDtypeStruct((_B, 1), f32),
                   jax.ShapeDtypeStruct((1, 1), f32)],
        scratch_shapes=[pltpu.VMEM((1, _SD), f32),
                        pltpu.VMEM((1, _SS), f32),
                        pltpu.VMEM((1, 1), f32)],
        **grid_spec,
    )(Xd, fo_g, so_g,
      fo_dense_W, fo_dense_b, fo_lin_W, fo_lin_b.reshape(1, 1),
      so_dense_W.reshape(1, _SD), so_dense_b.reshape(1, _SD),
      W1[:_SD], W1[_SD:], b1.reshape(1, 400),
      W2, b2.reshape(1, 400),
      W3.reshape(1, 400), b3.reshape(1, 1),
      msum, mexp, rexp)

    return rows + so_scalar


# speedup vs baseline: 2.9942x; 2.9885x over previous
import functools

import jax
import jax.numpy as jnp
from jax import lax
from jax.experimental import pallas as pl
from jax.experimental.pallas import tpu as pltpu
from jax.experimental.pallas import tpu_sc as plsc

_B = 4096
_ND = 13
_NS = 26
_F = _ND + _NS
_V = 100000
_E = 16
_MAXN = 0.1
_NW = 32
_R = _NS * _E
_RPW = _R // _NW
_BK = 512
_SD = _ND * _E


def _sc_gather(fo_tab, so_tab, xst):
    mesh = plsc.VectorSubcoreMesh(core_axis_name="c", subcore_axis_name="s")
    out_t = jax.ShapeDtypeStruct((_R, _B), jnp.float32)

    @functools.partial(
        pl.kernel,
        mesh=mesh,
        out_type=(out_t, out_t),
        compiler_params=pltpu.CompilerParams(use_tc_tiling_on_sc=False),
        scratch_types=[
            pltpu.VMEM((_B,), jnp.int32),
            pltpu.VMEM((_B,), jnp.float32),
            pltpu.VMEM((_B,), jnp.float32),
            pltpu.SemaphoreType.DMA,
            pltpu.SemaphoreType.DMA,
        ],
    )
    def k(fo_hbm, so_hbm, xst_hbm, gfo_out, gso_out, idx_v, a_v, b_v, s1, s2):
        wid = lax.axis_index("s") * 2 + lax.axis_index("c")
        r0 = wid * _RPW
        for j in range(_RPW):
            r = r0 + j
            f = r // _E
            pltpu.sync_copy(xst_hbm.at[f], idx_v)
            c1 = pltpu.async_copy(fo_hbm.at[r].at[idx_v], a_v, s1)
            c2 = pltpu.async_copy(so_hbm.at[r].at[idx_v], b_v, s2)
            c1.wait()
            c2.wait()
            pltpu.sync_copy(a_v, gfo_out.at[r])
            pltpu.sync_copy(b_v, gso_out.at[r])

    return k(fo_tab, so_tab, xst)


def _tc_body(xdt_ref, gfo_ref, gso_ref,
             fo_dw_ref, fo_db_ref, fo_lw_ref, fo_lb_ref,
             so_dwc_ref, so_dbc_ref,
             w1dt_ref, w1st_ref, b1c_ref, w2t_ref, b2c_ref, w3r_ref, b3_ref,
             m26_ref, mexp_ref, rexp_ref,
             row_out_ref, so_out_ref,
             cs_d, cs_s, ss_acc):
    i = pl.program_id(0)

    @pl.when(i == 0)
    def _():
        cs_d[...] = jnp.zeros_like(cs_d)
        cs_s[...] = jnp.zeros_like(cs_s)
        ss_acc[...] = jnp.zeros_like(ss_acc)

    xdt = xdt_ref[...]
    gfo = gfo_ref[...]
    gso = gso_ref[...]
    m26 = m26_ref[...]
    mexp = mexp_ref[...]
    rexp = rexp_ref[...]

    def renormed(g):
        n2 = jnp.dot(m26, g * g, preferred_element_type=jnp.float32)
        n = jnp.sqrt(n2)
        sc = jnp.where(n > _MAXN, _MAXN / (n + 1e-7), 1.0)
        return g * jnp.dot(mexp, sc, preferred_element_type=jnp.float32)

    gfo_s = renormed(gfo)
    gso_s = renormed(gso)

    fo_dw_sum = jnp.sum(fo_dw_ref[...], axis=1, keepdims=True)
    fo_db_sum = jnp.sum(fo_db_ref[...], axis=1, keepdims=True)
    fo_d = xdt * fo_dw_sum + fo_db_sum
    fo_srow = jnp.dot(m26, gfo_s, preferred_element_type=jnp.float32)
    lw = fo_lw_ref[...]
    first = (jnp.dot(lw[:, :_ND], fo_d, preferred_element_type=jnp.float32)
             + jnp.dot(lw[:, _ND:], fo_srow, preferred_element_type=jnp.float32)
             + fo_lb_ref[0, 0])

    so_d = (jnp.dot(rexp, xdt, preferred_element_type=jnp.float32)
            * so_dwc_ref[...] + so_dbc_ref[...])
    so_s = gso_s

    cs_d[...] += jnp.sum(so_d, axis=1, keepdims=True)
    cs_s[...] += jnp.sum(so_s, axis=1, keepdims=True)
    ss_acc[...] += (jnp.sum(so_d * so_d, keepdims=True)[:1, :1]
                    + jnp.sum(so_s * so_s, keepdims=True)[:1, :1])

    h = (jnp.dot(w1dt_ref[...], so_d, preferred_element_type=jnp.float32)
         + jnp.dot(w1st_ref[...], so_s, preferred_element_type=jnp.float32)
         + b1c_ref[...])
    h = jnp.maximum(h, 0.0)
    h = jnp.dot(w2t_ref[...], h, preferred_element_type=jnp.float32) + b2c_ref[...]
    h = jnp.maximum(h, 0.0)
    deep = jnp.dot(w3r_ref[...], h, preferred_element_type=jnp.float32) + b3_ref[0, 0]

    row_out_ref[...] = first + deep

    csd = cs_d[...]
    css = cs_s[...]
    so_out_ref[...] = 0.5 * (jnp.sum(csd * csd, keepdims=True)[:1, :1]
                             + jnp.sum(css * css, keepdims=True)[:1, :1]
                             - ss_acc[...])


def kernel(X, fo_dense_W, fo_dense_b, fo_tables, fo_lin_W, fo_lin_b,
           so_dense_W, so_dense_b, so_tables, W1, b1, W2, b2, W3, b3):
    f32 = jnp.float32
    XdT = X[:, :_ND].astype(f32).T
    XsT = X[:, _ND:].T

    fo_v = jnp.transpose(fo_tables, (0, 2, 1)).reshape(_R, _V)
    so_v = jnp.transpose(so_tables, (0, 2, 1)).reshape(_R, _V)

    gfo, gso = _sc_gather(fo_v, so_v, XsT)

    m26 = jnp.kron(jnp.eye(_NS, dtype=f32), jnp.ones((1, _E), f32))
    mexp = m26.T
    rexp = jnp.kron(jnp.eye(_ND, dtype=f32), jnp.ones((_E, 1), f32))

    nblk = _B // _BK
    whole = lambda shape: pl.BlockSpec(shape, lambda i: (0, 0))
    rows, so_scalar = pl.pallas_call(
        _tc_body,
        grid=(nblk,),
        in_specs=[
            pl.BlockSpec((_ND, _BK), lambda i: (0, i)),
            pl.BlockSpec((_R, _BK), lambda i: (0, i)),
            pl.BlockSpec((_R, _BK), lambda i: (0, i)),
            whole((_ND, _E)), whole((_ND, _E)),
            whole((1, _F)), whole((1, 1)),
            whole((_SD, 1)), whole((_SD, 1)),
            whole((400, _SD)), whole((400, _R)), whole((400, 1)),
            whole((400, 400)), whole((400, 1)),
            whole((1, 400)), whole((1, 1)),
            whole((_NS, _R)), whole((_R, _NS)), whole((_SD, _ND)),
        ],
        out_specs=[
            pl.BlockSpec((1, _BK), lambda i: (0, i)),
            pl.BlockSpec((1, 1), lambda i: (0, 0)),
        ],
        out_shape=[jax.ShapeDtypeStruct((1, _B), f32),
                   jax.ShapeDtypeStruct((1, 1), f32)],
        scratch_shapes=[pltpu.VMEM((_SD, 1), f32),
                        pltpu.VMEM((_R, 1), f32),
                        pltpu.VMEM((1, 1), f32)],
    )(XdT, gfo, gso,
      fo_dense_W, fo_dense_b, fo_lin_W, fo_lin_b.reshape(1, 1),
      so_dense_W.reshape(_SD, 1), so_dense_b.reshape(_SD, 1),
      W1[:_SD].T, W1[_SD:].T, b1.reshape(400, 1),
      W2.T, b2.reshape(400, 1),
      W3.reshape(1, 400), b3.reshape(1, 1),
      m26, mexp, rexp)

    return rows.reshape(_B, 1) + so_scalar


# speedup vs baseline: 3.3074x; 1.1046x over previous
import functools

import jax
import jax.numpy as jnp
from jax import lax
from jax.experimental import pallas as pl
from jax.experimental.pallas import tpu as pltpu
from jax.experimental.pallas import tpu_sc as plsc

_B = 4096
_ND = 13
_NS = 26
_F = _ND + _NS
_V = 100000
_E = 16
_MAXN = 0.1
_NW = 32
_R = _NS * _E
_RPW = _R // _NW
_BK = 512
_SD = _ND * _E


def _sc_gather(tab, xst):
    mesh = plsc.VectorSubcoreMesh(core_axis_name="c", subcore_axis_name="s")

    @functools.partial(
        pl.kernel,
        mesh=mesh,
        out_type=jax.ShapeDtypeStruct((_R, _B), jnp.float32),
        compiler_params=pltpu.CompilerParams(use_tc_tiling_on_sc=False),
        scratch_types=[
            pltpu.VMEM((2, _B), jnp.int32),
            pltpu.VMEM((_RPW, _B), jnp.float32),
            pltpu.SemaphoreType.DMA,
        ],
    )
    def k(tab_hbm, xst_hbm, g_out, idx2, g_v, s1):
        wid = lax.axis_index("s") * 2 + lax.axis_index("c")
        r0 = wid * _RPW
        f_lo = r0 // _E
        f_hi = (r0 + _RPW - 1) // _E
        pltpu.sync_copy(xst_hbm.at[f_lo], idx2.at[0])
        pltpu.sync_copy(xst_hbm.at[f_hi], idx2.at[1])
        copies = []
        for j in range(_RPW):
            r = r0 + j
            sel = r // _E - f_lo
            copies.append(
                pltpu.async_copy(tab_hbm.at[r].at[idx2.at[sel]], g_v.at[j], s1))
        for c in copies:
            c.wait()
        pltpu.sync_copy(g_v, g_out.at[pl.ds(r0, _RPW)])

    return k(tab, xst)


def _tc_body(xdt_ref, gfo_ref, gso_ref,
             fo_dw_ref, fo_db_ref, fo_lw_ref, fo_lb_ref,
             so_dwc_ref, so_dbc_ref,
             w1dt_ref, w1st_ref, b1c_ref, w2t_ref, b2c_ref, w3r_ref, b3_ref,
             m26_ref, mexp_ref, rexp_ref,
             row_out_ref, so_out_ref,
             cs_d, cs_s, ss_acc):
    i = pl.program_id(0)

    @pl.when(i == 0)
    def _():
        cs_d[...] = jnp.zeros_like(cs_d)
        cs_s[...] = jnp.zeros_like(cs_s)
        ss_acc[...] = jnp.zeros_like(ss_acc)

    xdt = xdt_ref[...]
    gfo = gfo_ref[...]
    gso = gso_ref[...]
    m26 = m26_ref[...]
    mexp = mexp_ref[...]
    rexp = rexp_ref[...]

    def renormed(g):
        n2 = jnp.dot(m26, g * g, preferred_element_type=jnp.float32)
        n = jnp.sqrt(n2)
        sc = jnp.where(n > _MAXN, _MAXN / (n + 1e-7), 1.0)
        return g * jnp.dot(mexp, sc, preferred_element_type=jnp.float32)

    gfo_s = renormed(gfo)
    gso_s = renormed(gso)

    fo_dw_sum = jnp.sum(fo_dw_ref[...], axis=1, keepdims=True)
    fo_db_sum = jnp.sum(fo_db_ref[...], axis=1, keepdims=True)
    fo_d = xdt * fo_dw_sum + fo_db_sum
    fo_srow = jnp.dot(m26, gfo_s, preferred_element_type=jnp.float32)
    lw = fo_lw_ref[...]
    first = (jnp.dot(lw[:, :_ND], fo_d, preferred_element_type=jnp.float32)
             + jnp.dot(lw[:, _ND:], fo_srow, preferred_element_type=jnp.float32)
             + fo_lb_ref[0, 0])

    so_d = (jnp.dot(rexp, xdt, preferred_element_type=jnp.float32)
            * so_dwc_ref[...] + so_dbc_ref[...])
    so_s = gso_s

    cs_d[...] += jnp.sum(so_d, axis=1, keepdims=True)
    cs_s[...] += jnp.sum(so_s, axis=1, keepdims=True)
    ss_acc[...] += (jnp.sum(so_d * so_d, keepdims=True)[:1, :1]
                    + jnp.sum(so_s * so_s, keepdims=True)[:1, :1])

    h = (jnp.dot(w1dt_ref[...], so_d, preferred_element_type=jnp.float32)
         + jnp.dot(w1st_ref[...], so_s, preferred_element_type=jnp.float32)
         + b1c_ref[...])
    h = jnp.maximum(h, 0.0)
    h = jnp.dot(w2t_ref[...], h, preferred_element_type=jnp.float32) + b2c_ref[...]
    h = jnp.maximum(h, 0.0)
    deep = jnp.dot(w3r_ref[...], h, preferred_element_type=jnp.float32) + b3_ref[0, 0]

    row_out_ref[...] = first + deep

    csd = cs_d[...]
    css = cs_s[...]
    so_out_ref[...] = 0.5 * (jnp.sum(csd * csd, keepdims=True)[:1, :1]
                             + jnp.sum(css * css, keepdims=True)[:1, :1]
                             - ss_acc[...])


def kernel(X, fo_dense_W, fo_dense_b, fo_tables, fo_lin_W, fo_lin_b,
           so_dense_W, so_dense_b, so_tables, W1, b1, W2, b2, W3, b3):
    f32 = jnp.float32
    XdT = X[:, :_ND].astype(f32).T
    XsT = X[:, _ND:].T

    fo_v = jnp.transpose(fo_tables, (0, 2, 1)).reshape(_R, _V)
    so_v = jnp.transpose(so_tables, (0, 2, 1)).reshape(_R, _V)

    gfo = _sc_gather(fo_v, XsT)
    gso = _sc_gather(so_v, XsT)

    m26 = jnp.kron(jnp.eye(_NS, dtype=f32), jnp.ones((1, _E), f32))
    mexp = m26.T
    rexp = jnp.kron(jnp.eye(_ND, dtype=f32), jnp.ones((_E, 1), f32))

    nblk = _B // _BK
    whole = lambda shape: pl.BlockSpec(shape, lambda i: (0, 0))
    rows, so_scalar = pl.pallas_call(
        _tc_body,
        grid=(nblk,),
        in_specs=[
            pl.BlockSpec((_ND, _BK), lambda i: (0, i)),
            pl.BlockSpec((_R, _BK), lambda i: (0, i)),
            pl.BlockSpec((_R, _BK), lambda i: (0, i)),
            whole((_ND, _E)), whole((_ND, _E)),
            whole((1, _F)), whole((1, 1)),
            whole((_SD, 1)), whole((_SD, 1)),
            whole((400, _SD)), whole((400, _R)), whole((400, 1)),
            whole((400, 400)), whole((400, 1)),
            whole((1, 400)), whole((1, 1)),
            whole((_NS, _R)), whole((_R, _NS)), whole((_SD, _ND)),
        ],
        out_specs=[
            pl.BlockSpec((1, _BK), lambda i: (0, i)),
            pl.BlockSpec((1, 1), lambda i: (0, 0)),
        ],
        out_shape=[jax.ShapeDtypeStruct((1, _B), f32),
                   jax.ShapeDtypeStruct((1, 1), f32)],
        scratch_shapes=[pltpu.VMEM((_SD, 1), f32),
                        pltpu.VMEM((_R, 1), f32),
                        pltpu.VMEM((1, 1), f32)],
    )(XdT, gfo, gso,
      fo_dense_W, fo_dense_b, fo_lin_W, fo_lin_b.reshape(1, 1),
      so_dense_W.reshape(_SD, 1), so_dense_b.reshape(_SD, 1),
      W1[:_SD].T, W1[_SD:].T, b1.reshape(400, 1),
      W2.T, b2.reshape(400, 1),
      W3.reshape(1, 400), b3.reshape(1, 1),
      m26, mexp, rexp)

    return rows.reshape(_B, 1) + so_scalar
